# baseline (speedup 1.0000x reference)
import functools

import jax
import jax.numpy as jnp
from jax import lax
from jax.experimental import pallas as pl
from jax.experimental.pallas import tpu as pltpu
from jax.experimental.pallas import tpu_sc as plsc

N_NODES = 10000
N_EDGES = 320000
EDGE_CHUNK = 100
DEPTH = 3
NUM_WORKERS = 32
SUBCORES = 16


def _sc_segment_add(h, src2, dst2):
    n, f = h.shape
    nblk = src2.shape[0] // NUM_WORKERS
    sb = 25
    nsup = nblk // sb
    rz = n // SUBCORES
    nzb = (n + EDGE_CHUNK - 1) // EDGE_CHUNK
    mesh = plsc.VectorSubcoreMesh(core_axis_name="c", subcore_axis_name="s")

    @functools.partial(
        pl.kernel,
        mesh=mesh,
        out_type=jax.ShapeDtypeStruct((2, n, f), jnp.float32),
        compiler_params=pltpu.CompilerParams(use_tc_tiling_on_sc=False),
        scratch_types=[
            pltpu.VMEM_SHARED((n, f), jnp.float32),
            pltpu.VMEM((sb, EDGE_CHUNK), jnp.int32),
            pltpu.VMEM((sb, EDGE_CHUNK), jnp.int32),
            pltpu.VMEM((EDGE_CHUNK, f), jnp.float32),
            pltpu.VMEM((EDGE_CHUNK, f), jnp.float32),
            pltpu.VMEM((EDGE_CHUNK, f), jnp.float32),
            pltpu.SemaphoreType.DMA,
            pltpu.SemaphoreType.DMA,
            pltpu.SemaphoreType.DMA,
            pltpu.SemaphoreType.DMA,
            pltpu.SemaphoreType.DMA,
            pltpu.SemaphoreType.DMA,
            pltpu.SemaphoreType.DMA,
            pltpu.SemaphoreType.DMA,
        ],
    )
    def k(h_hbm, src_hbm, dst_hbm, out_hbm, acc,
          src_v, dst_v, rows0, rows1, rows2,
          g0, g1, g2, g3, s0, s1, s2, s3):
        c = lax.axis_index("c")
        s = lax.axis_index("s")
        wid = c * SUBCORES + s

        bufs = (rows0, rows1, rows2)
        gsems = (g0, g1, g2, g3)
        ssems = (s0, s1, s2, s3)

        @pl.loop(0, EDGE_CHUNK)
        def _(r):
            @pl.loop(0, f // 16)
            def _(q):
                rows0[r, pl.ds(q * 16, 16)] = jnp.zeros((16,), jnp.float32)

        nzi = (nzb + SUBCORES - 1) // SUBCORES

        @pl.loop(0, (nzi + 3) // 4)
        def _(gg):
            for q in range(4):
                blk = s + SUBCORES * (gg * 4 + q)

                @pl.when(blk < nzb)
                def _():
                    pltpu.async_copy(
                        rows0, acc.at[pl.ds(blk * EDGE_CHUNK, EDGE_CHUNK)],
                        gsems[q])
            for q in range(4):
                blk = s + SUBCORES * (gg * 4 + q)

                @pl.when(blk < nzb)
                def _():
                    pltpu.make_async_copy(
                        rows0, acc.at[pl.ds(blk * EDGE_CHUNK, EDGE_CHUNK)],
                        gsems[q]).wait()

        plsc.subcore_barrier()

        def start_g(j, buf, sem):
            pltpu.async_copy(h_hbm.at[src_v.at[j]], buf, sem)

        def wait_g(j, buf, sem):
            pltpu.make_async_copy(h_hbm.at[src_v.at[j]], buf, sem).wait()

        def start_s(j, buf, sem):
            pltpu.async_copy(buf, acc.at[dst_v.at[j]], sem, add=True)

        def wait_s(j, buf, sem):
            pltpu.make_async_copy(buf, acc.at[dst_v.at[j]], sem).wait()

        @pl.loop(0, nsup)
        def _(t):
            base = wid * nblk + t * sb
            pltpu.sync_copy(src_hbm.at[pl.ds(base, sb)], src_v)
            pltpu.sync_copy(dst_hbm.at[pl.ds(base, sb)], dst_v)
            for q in range(DEPTH):
                start_g(q, bufs[q], gsems[q])

            @pl.loop(0, (sb - 1) // DEPTH)
            def _(p):
                j0 = DEPTH * p
                for q in range(DEPTH):
                    wait_g(j0 + q, bufs[q], gsems[q])
                    start_s(j0 + q, bufs[q], ssems[q])
                for q in range(DEPTH):
                    wait_s(j0 + q, bufs[q], ssems[q])
                    jn = j0 + DEPTH + q

                    @pl.when(jn < sb)
                    def _():
                        start_g(jn, bufs[q], gsems[q])

            qt = (sb - 1) % DEPTH
            wait_g(sb - 1, bufs[qt], gsems[qt])
            pltpu.sync_copy(bufs[qt], acc.at[dst_v.at[sb - 1]], add=True)

        plsc.subcore_barrier()
        pltpu.sync_copy(acc.at[pl.ds(s * rz, rz)], out_hbm.at[c, pl.ds(s * rz, rz)])

    return k(h, src2, dst2)


_TC_PARAMS = pltpu.CompilerParams(vmem_limit_bytes=60 * 1024 * 1024)


def _project0(x, cdk, W0a):
    n, fx = x.shape
    fc = cdk.shape[1]
    hdim = W0a.shape[1]

    def body(x_ref, c_ref, wa_ref, o_ref):
        o_ref[...] = (
            jnp.dot(x_ref[...], wa_ref[0:fx, :],
                    preferred_element_type=jnp.float32)
            + jnp.dot(c_ref[...], wa_ref[fx:fx + fc, :],
                      preferred_element_type=jnp.float32))

    return pl.pallas_call(
        body,
        out_shape=jax.ShapeDtypeStruct((n, hdim), jnp.float32),
        compiler_params=_TC_PARAMS,
    )(x, cdk, W0a)


def _dense_mid(u, agg, ba, Wb, bb, g, be, Wnext):
    n, hdim = u.shape

    def body(u_ref, agg_ref, ba_ref, wb_ref, bb_ref, g_ref, be_ref, wn_ref,
             o_ref):
        t = jnp.maximum(u_ref[...] + agg_ref[0] + agg_ref[1] + ba_ref[...],
                        0.0)
        m2 = (jnp.dot(t, wb_ref[...], preferred_element_type=jnp.float32)
              + bb_ref[...])
        mean = jnp.mean(m2, axis=0, keepdims=True)
        var = jnp.mean(m2 * m2, axis=0, keepdims=True) - mean * mean
        scale = g_ref[...] * lax.rsqrt(var + 1e-5)
        shift = be_ref[...] - mean * scale
        v = m2 * scale + shift
        h1 = jnp.where(v > 0, v, jnp.exp(jnp.minimum(v, 0.0)) - 1.0)
        o_ref[...] = jnp.dot(h1, wn_ref[...],
                             preferred_element_type=jnp.float32)

    return pl.pallas_call(
        body,
        out_shape=jax.ShapeDtypeStruct((n, Wnext.shape[1]), jnp.float32),
        compiler_params=_TC_PARAMS,
    )(u, agg, ba.reshape(1, -1), Wb, bb.reshape(1, -1),
      g.reshape(1, -1), be.reshape(1, -1), Wnext)


def _dense_head(u, agg, ba, Wb, bb, g, be, Wlin, blin):
    n, hdim = u.shape

    def body(u_ref, agg_ref, ba_ref, wb_ref, bb_ref, g_ref, be_ref,
             w_ref, b_ref, o_ref):
        t = jnp.maximum(u_ref[...] + agg_ref[0] + agg_ref[1] + ba_ref[...],
                        0.0)
        m2 = (jnp.dot(t, wb_ref[...], preferred_element_type=jnp.float32)
              + bb_ref[...])
        mean = jnp.mean(m2, axis=0, keepdims=True)
        var = jnp.mean(m2 * m2, axis=0, keepdims=True) - mean * mean
        scale = g_ref[...] * lax.rsqrt(var + 1e-5)
        shift = be_ref[...] - mean * scale
        v = m2 * scale + shift
        v = jnp.where(v > 0, v, jnp.exp(jnp.minimum(v, 0.0)) - 1.0)
        logit = jnp.sum(v * w_ref[...], axis=1, keepdims=True) + b_ref[...]
        o_ref[...] = 1.0 / (1.0 + jnp.exp(-logit))

    return pl.pallas_call(
        body,
        out_shape=jax.ShapeDtypeStruct((n, 1), jnp.float32),
        compiler_params=_TC_PARAMS,
    )(u, agg, ba.reshape(1, -1), Wb, bb.reshape(1, -1),
      g.reshape(1, -1), be.reshape(1, -1),
      Wlin.reshape(1, -1), blin.reshape(1, 1))


def kernel(x, cdk_desc, edge_index, W0a, b0a, W0b, b0b, g0, be0,
           W1a, b1a, W1b, b1b, g1, be1, Wlin, blin):
    src2 = edge_index[0].reshape(N_EDGES // EDGE_CHUNK, EDGE_CHUNK)
    dst2 = edge_index[1].reshape(N_EDGES // EDGE_CHUNK, EDGE_CHUNK)

    u0 = _project0(x, cdk_desc, W0a)
    agg0 = _sc_segment_add(u0, src2, dst2)
    u1 = _dense_mid(u0, agg0, b0a, W0b, b0b, g0, be0, W1a)
    agg1 = _sc_segment_add(u1, src2, dst2)
    out = _dense_head(u1, agg1, b1a, W1b, b1b, g1, be1, Wlin, blin)
    return out.reshape(-1)


# speedup vs baseline: 1.0775x; 1.0775x over previous
import functools

import jax
import jax.numpy as jnp
from jax import lax
from jax.experimental import pallas as pl
from jax.experimental.pallas import tpu as pltpu
from jax.experimental.pallas import tpu_sc as plsc

N_NODES = 10000
N_EDGES = 320000
EDGE_CHUNK = 80
NUM_WORKERS = 32
SUBCORES = 16


def _sc_segment_add(h, src2, dst2):
    n, f = h.shape
    nblk = src2.shape[0] // NUM_WORKERS
    sb = 25
    nsup = nblk // sb
    rz = n // SUBCORES
    nzb = (n + EDGE_CHUNK - 1) // EDGE_CHUNK
    mesh = plsc.VectorSubcoreMesh(core_axis_name="c", subcore_axis_name="s")

    @functools.partial(
        pl.kernel,
        mesh=mesh,
        out_type=jax.ShapeDtypeStruct((2, n, f), jnp.float32),
        compiler_params=pltpu.CompilerParams(use_tc_tiling_on_sc=False),
        scratch_types=[
            pltpu.VMEM_SHARED((n, f), jnp.float32),
            pltpu.VMEM((sb, EDGE_CHUNK), jnp.int32),
            pltpu.VMEM((sb, EDGE_CHUNK), jnp.int32),
            pltpu.VMEM((EDGE_CHUNK, f), jnp.float32),
            pltpu.VMEM((EDGE_CHUNK, f), jnp.float32),
            pltpu.VMEM((EDGE_CHUNK, f), jnp.float32),
            pltpu.VMEM((EDGE_CHUNK, f), jnp.float32),
            pltpu.SemaphoreType.DMA,
            pltpu.SemaphoreType.DMA,
            pltpu.SemaphoreType.DMA,
            pltpu.SemaphoreType.DMA,
            pltpu.SemaphoreType.DMA,
            pltpu.SemaphoreType.DMA,
            pltpu.SemaphoreType.DMA,
            pltpu.SemaphoreType.DMA,
        ],
    )
    def k(h_hbm, src_hbm, dst_hbm, out_hbm, acc,
          src_v, dst_v, rows0, rows1, rows2, rows3,
          g0, g1, g2, g3, s0, s1, s2, s3):
        c = lax.axis_index("c")
        s = lax.axis_index("s")
        wid = c * SUBCORES + s

        bufs = (rows0, rows1, rows2, rows3)
        gsems = (g0, g1, g2, g3)
        ssems = (s0, s1, s2, s3)

        @pl.loop(0, EDGE_CHUNK)
        def _(r):
            @pl.loop(0, f // 16)
            def _(q):
                rows0[r, pl.ds(q * 16, 16)] = jnp.zeros((16,), jnp.float32)

        nzi = (nzb + SUBCORES - 1) // SUBCORES

        @pl.loop(0, (nzi + 3) // 4)
        def _(gg):
            for q in range(4):
                blk = s + SUBCORES * (gg * 4 + q)

                @pl.when(blk < nzb)
                def _():
                    pltpu.async_copy(
                        rows0, acc.at[pl.ds(blk * EDGE_CHUNK, EDGE_CHUNK)],
                        gsems[q])
            for q in range(4):
                blk = s + SUBCORES * (gg * 4 + q)

                @pl.when(blk < nzb)
                def _():
                    pltpu.make_async_copy(
                        rows0, acc.at[pl.ds(blk * EDGE_CHUNK, EDGE_CHUNK)],
                        gsems[q]).wait()

        plsc.subcore_barrier()

        def start_g(j, buf, sem):
            pltpu.async_copy(h_hbm.at[src_v.at[j]], buf, sem)

        def wait_g(j, buf, sem):
            pltpu.make_async_copy(h_hbm.at[src_v.at[j]], buf, sem).wait()

        def start_s(j, buf, sem):
            pltpu.async_copy(buf, acc.at[dst_v.at[j]], sem, add=True)

        def wait_s(j, buf, sem):
            pltpu.make_async_copy(buf, acc.at[dst_v.at[j]], sem).wait()

        @pl.loop(0, nsup)
        def _(t):
            base = wid * nblk + t * sb
            pltpu.sync_copy(src_hbm.at[pl.ds(base, sb)], src_v)
            pltpu.sync_copy(dst_hbm.at[pl.ds(base, sb)], dst_v)
            for q in range(4):
                start_g(q, bufs[q], gsems[q])

            @pl.loop(0, (sb - 1) // 4)
            def _(p):
                j0 = 4 * p
                for q in range(4):
                    wait_g(j0 + q, bufs[q], gsems[q])
                    start_s(j0 + q, bufs[q], ssems[q])
                for q in range(4):
                    wait_s(j0 + q, bufs[q], ssems[q])
                    jn = j0 + 4 + q

                    @pl.when(jn < sb)
                    def _():
                        start_g(jn, bufs[q], gsems[q])

            wait_g(sb - 1, rows0, g0)
            pltpu.sync_copy(rows0, acc.at[dst_v.at[sb - 1]], add=True)

        plsc.subcore_barrier()
        pltpu.sync_copy(acc.at[pl.ds(s * rz, rz)], out_hbm.at[c, pl.ds(s * rz, rz)])

    return k(h, src2, dst2)


_TC_PARAMS = pltpu.CompilerParams(vmem_limit_bytes=60 * 1024 * 1024)


def _project0(x, cdk, W0a):
    n, fx = x.shape
    fc = cdk.shape[1]
    hdim = W0a.shape[1]

    def body(x_ref, c_ref, wa_ref, o_ref):
        o_ref[...] = (
            jnp.dot(x_ref[...], wa_ref[0:fx, :],
                    preferred_element_type=jnp.float32)
            + jnp.dot(c_ref[...], wa_ref[fx:fx + fc, :],
                      preferred_element_type=jnp.float32))

    return pl.pallas_call(
        body,
        out_shape=jax.ShapeDtypeStruct((n, hdim), jnp.float32),
        compiler_params=_TC_PARAMS,
    )(x, cdk, W0a)


def _dense_mid(u, agg, ba, Wb, bb, g, be, Wnext):
    n, hdim = u.shape

    def body(u_ref, agg_ref, ba_ref, wb_ref, bb_ref, g_ref, be_ref, wn_ref,
             o_ref):
        t = jnp.maximum(u_ref[...] + agg_ref[0] + agg_ref[1] + ba_ref[...],
                        0.0)
        m2 = (jnp.dot(t, wb_ref[...], preferred_element_type=jnp.float32)
              + bb_ref[...])
        mean = jnp.mean(m2, axis=0, keepdims=True)
        var = jnp.mean(m2 * m2, axis=0, keepdims=True) - mean * mean
        scale = g_ref[...] * lax.rsqrt(var + 1e-5)
        shift = be_ref[...] - mean * scale
        v = m2 * scale + shift
        h1 = jnp.where(v > 0, v, jnp.exp(jnp.minimum(v, 0.0)) - 1.0)
        o_ref[...] = jnp.dot(h1, wn_ref[...],
                             preferred_element_type=jnp.float32)

    return pl.pallas_call(
        body,
        out_shape=jax.ShapeDtypeStruct((n, Wnext.shape[1]), jnp.float32),
        compiler_params=_TC_PARAMS,
    )(u, agg, ba.reshape(1, -1), Wb, bb.reshape(1, -1),
      g.reshape(1, -1), be.reshape(1, -1), Wnext)


def _dense_head(u, agg, ba, Wb, bb, g, be, Wlin, blin):
    n, hdim = u.shape

    def body(u_ref, agg_ref, ba_ref, wb_ref, bb_ref, g_ref, be_ref,
             w_ref, b_ref, o_ref):
        t = jnp.maximum(u_ref[...] + agg_ref[0] + agg_ref[1] + ba_ref[...],
                        0.0)
        m2 = (jnp.dot(t, wb_ref[...], preferred_element_type=jnp.float32)
              + bb_ref[...])
        mean = jnp.mean(m2, axis=0, keepdims=True)
        var = jnp.mean(m2 * m2, axis=0, keepdims=True) - mean * mean
        scale = g_ref[...] * lax.rsqrt(var + 1e-5)
        shift = be_ref[...] - mean * scale
        v = m2 * scale + shift
        v = jnp.where(v > 0, v, jnp.exp(jnp.minimum(v, 0.0)) - 1.0)
        logit = jnp.sum(v * w_ref[...], axis=1, keepdims=True) + b_ref[...]
        o_ref[...] = 1.0 / (1.0 + jnp.exp(-logit))

    return pl.pallas_call(
        body,
        out_shape=jax.ShapeDtypeStruct((n, 1), jnp.float32),
        compiler_params=_TC_PARAMS,
    )(u, agg, ba.reshape(1, -1), Wb, bb.reshape(1, -1),
      g.reshape(1, -1), be.reshape(1, -1),
      Wlin.reshape(1, -1), blin.reshape(1, 1))


def kernel(x, cdk_desc, edge_index, W0a, b0a, W0b, b0b, g0, be0,
           W1a, b1a, W1b, b1b, g1, be1, Wlin, blin):
    src2 = edge_index[0].reshape(N_EDGES // EDGE_CHUNK, EDGE_CHUNK)
    dst2 = edge_index[1].reshape(N_EDGES // EDGE_CHUNK, EDGE_CHUNK)

    u0 = _project0(x, cdk_desc, W0a)
    agg0 = _sc_segment_add(u0, src2, dst2)
    u1 = _dense_mid(u0, agg0, b0a, W0b, b0b, g0, be0, W1a)
    agg1 = _sc_segment_add(u1, src2, dst2)
    out = _dense_head(u1, agg1, b1a, W1b, b1b, g1, be1, Wlin, blin)
    return out.reshape(-1)
